# baseline (speedup 1.0000x reference)
import functools

import jax
import jax.numpy as jnp
from jax.experimental import pallas as pl
from jax.experimental.pallas import tpu as pltpu

_CHOICE_TEMPERATURE = 4.5
_INT_MIN = -2147483648


def _body(temp_ref, ml_ref, la_ref, lb_ref, vidx_ref, gumbel_ref, mask_ref,
          zpred_ref, maskbc_ref, conf_ref, *, s_blk, s, v, nb, nj):
    bi = pl.program_id(0)
    j = pl.program_id(1)
    h = v // 2
    xa = la_ref[0]
    xb = lb_ref[0]
    m = jnp.maximum(jnp.max(xa, axis=-1), jnp.max(xb, axis=-1))
    ca = jnp.where(xa == m[:, None], vidx_ref[:, :h], v)
    cb = jnp.where(xb == m[:, None], vidx_ref[:, h:], v)
    amax = jnp.minimum(jnp.min(ca, axis=-1), jnp.min(cb, axis=-1))
    se = jnp.sum(jnp.exp(xa - m[:, None]), axis=-1) + \
        jnp.sum(jnp.exp(xb - m[:, None]), axis=-1)
    pmax = 1.0 / se
    temp = temp_ref[0]
    g = gumbel_ref[0, 0]
    mk = mask_ref[0, 0]
    conf = jnp.where(mk != 0, pmax + temp * g, jnp.inf)
    zpred_ref[0, 0, :] = amax
    conf_ref[0, pl.ds(bi * s + j * s_blk, s_blk)] = conf

    @pl.when((bi == nb - 1) & (j == nj - 1))
    def _rank():
        c = conf_ref[0, :].reshape(nb, s)
        cbits = jax.lax.bitcast_convert_type(c, jnp.int32)
        key = jnp.where(cbits < 0, cbits ^ 0x7FFFFFFF, cbits)
        k = ml_ref[0]
        imin = jnp.int32(_INT_MIN)

        def bit1(i, res_u):
            cand_u = res_u | (jnp.int32(1) << (31 - i))
            cnt = jnp.sum((key < (cand_u ^ imin)).astype(jnp.int32),
                          axis=1, keepdims=True)
            return jnp.where(cnt < k, cand_u, res_u)

        res_u = jax.lax.fori_loop(0, 32, bit1, jnp.zeros((nb, 1), jnp.int32))
        t_s = res_u ^ imin
        lt = key < t_s
        eq = key == t_s
        jrem = k - jnp.sum(lt.astype(jnp.int32), axis=1, keepdims=True)
        idx = vidx_ref[:, :s]

        def bit2(i, res2):
            cand2 = res2 | (jnp.int32(1) << (9 - i))
            cnt = jnp.sum((eq & (idx < cand2)).astype(jnp.int32),
                          axis=1, keepdims=True)
            return jnp.where(cnt < jrem, cand2, res2)

        t_idx = jax.lax.fori_loop(0, 10, bit2, jnp.zeros((nb, 1), jnp.int32))
        maskbc_ref[:, :] = (lt | (eq & (idx <= t_idx))).astype(jnp.int32)


def kernel(logits, ratio, gumbel, z_indices, mask, mask_num):
    del z_indices
    b, s, v = logits.shape
    s_blk = 512
    nj = s // s_blk
    h = v // 2

    r = ratio[0]
    mask_ratio = jnp.cos(r * jnp.pi / 2.0)
    mask_len = jnp.maximum(jnp.ceil(mask_num * mask_ratio), 1.0).astype(jnp.int32)
    temperature = (_CHOICE_TEMPERATURE * (1.0 - mask_ratio)).astype(jnp.float32)

    vidx = jnp.arange(v, dtype=jnp.int32).reshape(1, v)
    gumbel3 = gumbel.reshape(b * nj, 1, s_blk)
    mask3 = mask.astype(jnp.int32).reshape(b * nj, 1, s_blk)

    zpred, maskbc = pl.pallas_call(
        functools.partial(_body, s_blk=s_blk, s=s, v=v, nb=b, nj=nj),
        grid=(b, nj),
        in_specs=[
            pl.BlockSpec(memory_space=pltpu.SMEM),
            pl.BlockSpec(memory_space=pltpu.SMEM),
            pl.BlockSpec((1, s_blk, h), lambda bi, ji: (bi, ji, 0)),
            pl.BlockSpec((1, s_blk, h), lambda bi, ji: (bi, ji, 1)),
            pl.BlockSpec((1, v), lambda bi, ji: (0, 0)),
            pl.BlockSpec((1, 1, s_blk), lambda bi, ji: (bi * nj + ji, 0, 0)),
            pl.BlockSpec((1, 1, s_blk), lambda bi, ji: (bi * nj + ji, 0, 0)),
        ],
        out_specs=[
            pl.BlockSpec((1, 1, s_blk), lambda bi, ji: (bi * nj + ji, 0, 0)),
            pl.BlockSpec((b, s), lambda bi, ji: (0, 0)),
        ],
        out_shape=[
            jax.ShapeDtypeStruct((b * nj, 1, s_blk), jnp.int32),
            jax.ShapeDtypeStruct((b, s), jnp.int32),
        ],
        scratch_shapes=[pltpu.VMEM((1, b * s), jnp.float32)],
    )(temperature.reshape(1), mask_len.reshape(1), logits, logits, vidx,
      gumbel3, mask3)

    return zpred.reshape(b, s), maskbc.astype(jnp.bool_)


# speedup vs baseline: 1.1625x; 1.1625x over previous
import functools

import jax
import jax.numpy as jnp
from jax import lax
from jax.experimental import pallas as pl
from jax.experimental.pallas import tpu as pltpu
from jax.experimental.pallas import tpu_sc as plsc

_CHOICE_TEMPERATURE = 4.5
_INT_MIN = -2147483648
_GDN = lax.GatherDimensionNumbers(offset_dims=(), collapsed_slice_dims=(0,),
                                  start_index_map=(0,))


def _tc_body(temp_ref, logits_ref, vidx_ref, gumbel_ref, mask_ref,
             zpred_ref, conf_ref, *, v):
    x = logits_ref[0]
    m = jnp.max(x, axis=-1)
    cand = jnp.where(x == m[:, None], vidx_ref[:], v)
    amax = jnp.min(cand, axis=-1)
    se = jnp.sum(jnp.exp(x - m[:, None]), axis=-1)
    conf = jnp.where(mask_ref[0, 0] != 0,
                     1.0 / se + temp_ref[0] * gumbel_ref[0, 0], jnp.inf)
    zpred_ref[0, 0, :] = amax
    conf_ref[0, 0, :] = conf


def _sc_body(conf_hbm, k_hbm, out_hbm, conf_v, key_v, out_v, k_v, *, s, nb):
    wid = lax.axis_index("s") * 2 + lax.axis_index("c")
    base = wid * s
    nch = s // 16

    @pl.when(wid < nb)
    def _():
        pltpu.sync_copy(conf_hbm.at[pl.ds(base, s)], conf_v)
        pltpu.sync_copy(k_hbm, k_v)

    kv = k_v[...]
    zero = jnp.zeros((16,), jnp.int32)
    one = jnp.full((16,), 1, jnp.int32)
    imin = jnp.full((16,), _INT_MIN, jnp.int32)
    iota = lax.broadcasted_iota(jnp.int32, (16,), 0)
    sixteen = jnp.full((16,), 16, jnp.int32)

    def lanesum(v16):
        for sh in (8, 4, 2, 1):
            perm = (iota + sh) & 15
            v16 = v16 + lax.gather(v16, perm[:, None], _GDN, (1,),
                                   mode=lax.GatherScatterMode.PROMISE_IN_BOUNDS)
        return v16

    def mkkey(i, carry):
        c16 = conf_v[pl.ds(i * 16, 16)]
        b16 = jax.lax.bitcast_convert_type(c16, jnp.int32)
        key_v[pl.ds(i * 16, 16)] = jnp.where(b16 < 0, b16 ^ 0x7FFFFFFF, b16)
        return carry

    lax.fori_loop(0, nch, mkkey, jnp.int32(0))

    def count_lt(thr):
        def cbody(i, cnt):
            k16 = key_v[pl.ds(i * 16, 16)]
            return cnt + jnp.where(k16 < thr, one, zero)
        return lanesum(lax.fori_loop(0, nch, cbody, zero))

    def bit1(i, carry):
        res_u, bitv = carry
        cand_u = res_u | bitv
        cnt = count_lt(cand_u ^ imin)
        return (jnp.where(cnt < kv, cand_u, res_u),
                lax.shift_right_logical(bitv, one))

    res_u, _ = lax.fori_loop(0, 32, bit1, (zero, imin))
    t_s = res_u ^ imin
    jrem = kv - count_lt(t_s)

    def count_eq_lt(cap):
        def cbody(i, carry):
            cnt, idx16 = carry
            k16 = key_v[pl.ds(i * 16, 16)]
            m16 = (k16 == t_s) & (idx16 < cap)
            return (cnt + jnp.where(m16, one, zero), idx16 + sixteen)
        return lanesum(lax.fori_loop(0, nch, cbody, (zero, iota))[0])

    def bit2(i, carry):
        res2, bitv = carry
        cand2 = res2 | bitv
        cnt = count_eq_lt(cand2)
        return (jnp.where(cnt < jrem, cand2, res2),
                lax.shift_right_logical(bitv, one))

    t_idx, _ = lax.fori_loop(0, 10, bit2,
                             (zero, jnp.full((16,), 512, jnp.int32)))

    def emit(i, carry):
        idx16 = carry
        k16 = key_v[pl.ds(i * 16, 16)]
        m16 = (k16 < t_s) | ((k16 == t_s) & (idx16 <= t_idx))
        out_v[pl.ds(i * 16, 16)] = jnp.where(m16, one, zero)
        return idx16 + sixteen

    lax.fori_loop(0, nch, emit, iota)

    @pl.when(wid < nb)
    def _():
        pltpu.sync_copy(out_v, out_hbm.at[pl.ds(base, s)])


def kernel(logits, ratio, gumbel, z_indices, mask, mask_num):
    del z_indices
    b, s, v = logits.shape
    s_blk = 512
    nj = s // s_blk

    r = ratio[0]
    mask_ratio = jnp.cos(r * jnp.pi / 2.0)
    mask_len = jnp.maximum(jnp.ceil(mask_num * mask_ratio), 1.0).astype(jnp.int32)
    temperature = (_CHOICE_TEMPERATURE * (1.0 - mask_ratio)).astype(jnp.float32)

    vidx = jnp.arange(v, dtype=jnp.int32).reshape(1, v)
    gumbel3 = gumbel.reshape(b * nj, 1, s_blk)
    mask3 = mask.astype(jnp.int32).reshape(b * nj, 1, s_blk)

    zpred, conf = pl.pallas_call(
        functools.partial(_tc_body, v=v),
        grid=(b, nj),
        in_specs=[
            pl.BlockSpec(memory_space=pltpu.SMEM),
            pl.BlockSpec((1, s_blk, v), lambda bi, ji: (bi, ji, 0)),
            pl.BlockSpec((1, v), lambda bi, ji: (0, 0)),
            pl.BlockSpec((1, 1, s_blk), lambda bi, ji: (bi * nj + ji, 0, 0)),
            pl.BlockSpec((1, 1, s_blk), lambda bi, ji: (bi * nj + ji, 0, 0)),
        ],
        out_specs=[
            pl.BlockSpec((1, 1, s_blk), lambda bi, ji: (bi * nj + ji, 0, 0)),
            pl.BlockSpec((1, 1, s_blk), lambda bi, ji: (bi * nj + ji, 0, 0)),
        ],
        out_shape=[
            jax.ShapeDtypeStruct((b * nj, 1, s_blk), jnp.int32),
            jax.ShapeDtypeStruct((b * nj, 1, s_blk), jnp.float32),
        ],
    )(temperature.reshape(1), logits, vidx, gumbel3, mask3)

    conf1 = conf.reshape(b * s)
    kvec = jnp.broadcast_to(mask_len, (16,)).astype(jnp.int32)

    sc_topk = functools.partial(
        pl.kernel,
        mesh=plsc.VectorSubcoreMesh(core_axis_name="c", subcore_axis_name="s"),
        out_type=jax.ShapeDtypeStruct((b * s,), jnp.int32),
        scratch_types=[
            pltpu.VMEM((s,), jnp.float32),
            pltpu.VMEM((s,), jnp.int32),
            pltpu.VMEM((s,), jnp.int32),
            pltpu.VMEM((16,), jnp.int32),
        ],
    )(functools.partial(_sc_body, s=s, nb=b))

    maskbc = sc_topk(conf1, kvec).reshape(b, s)
    return zpred.reshape(b, s), maskbc.astype(jnp.bool_)
